# Initial kernel scaffold; baseline (speedup 1.0000x reference)
import jax
import jax.numpy as jnp
from jax.experimental import pallas as pl


def kernel(verts, faces, points):
    raise NotImplementedError("write your pallas kernel here")


# speedup vs baseline: 2.7374x; 2.7374x over previous
import jax
import jax.numpy as jnp
from jax.experimental import pallas as pl
from jax.experimental.pallas import tpu as pltpu

EPS_ = 1e-12
NT = 256
FC = 1024


def _mesh_sdf_kernel(tri_ref, pts_ref, dist_ref, assoc_ref, loss_ref):
    F = tri_ref.shape[1]
    n_chunks = F // FC

    px = pts_ref[:, 0:1]
    py = pts_ref[:, 1:2]
    pz = pts_ref[:, 2:3]

    best_d = jnp.full((NT, 1), jnp.inf, dtype=jnp.float32)
    best_i = jnp.zeros((NT, 1), dtype=jnp.int32)

    for j in range(n_chunks):
        sl = slice(j * FC, (j + 1) * FC)
        ax = tri_ref[0:1, sl]
        ay = tri_ref[1:2, sl]
        az = tri_ref[2:3, sl]
        bx = tri_ref[3:4, sl]
        by = tri_ref[4:5, sl]
        bz = tri_ref[5:6, sl]
        cx = tri_ref[6:7, sl]
        cy = tri_ref[7:8, sl]
        cz = tri_ref[8:9, sl]

        abx = bx - ax
        aby = by - ay
        abz = bz - az
        acx = cx - ax
        acy = cy - ay
        acz = cz - az
        bcx = cx - bx
        bcy = cy - by
        bcz = cz - bz

        apx = px - ax
        apy = py - ay
        apz = pz - az
        d1 = abx * apx + aby * apy + abz * apz
        d2 = acx * apx + acy * apy + acz * apz
        bpx = px - bx
        bpy = py - by
        bpz = pz - bz
        d3 = abx * bpx + aby * bpy + abz * bpz
        d4 = acx * bpx + acy * bpy + acz * bpz
        cpx = px - cx
        cpy = py - cy
        cpz = pz - cz
        d5 = abx * cpx + aby * cpy + abz * cpz
        d6 = acx * cpx + acy * cpy + acz * cpz

        vc = d1 * d4 - d3 * d2
        vb = d5 * d2 - d1 * d6
        va = d3 * d6 - d5 * d4

        denom = (va + vb) + vc
        denom = jnp.where(jnp.abs(denom) < EPS_, EPS_, denom)
        v = vb / denom
        w = vc / denom
        rx = ax + v * abx + w * acx
        ry = ay + v * aby + w * acy
        rz = az + v * abz + w * acz

        e45 = d4 - d3
        e56 = d5 - d6
        w_bc = e45 / jnp.maximum(e45 + e56, EPS_)
        m_bc = (va <= 0) & (e45 >= 0) & (e56 >= 0)
        rx = jnp.where(m_bc, bx + w_bc * bcx, rx)
        ry = jnp.where(m_bc, by + w_bc * bcy, ry)
        rz = jnp.where(m_bc, bz + w_bc * bcz, rz)

        v_ac = d2 / jnp.maximum(d2 - d6, EPS_)
        m_ac = (vb <= 0) & (d2 >= 0) & (d6 <= 0)
        rx = jnp.where(m_ac, ax + v_ac * acx, rx)
        ry = jnp.where(m_ac, ay + v_ac * acy, ry)
        rz = jnp.where(m_ac, az + v_ac * acz, rz)

        v_ab = d1 / jnp.maximum(d1 - d3, EPS_)
        m_ab = (vc <= 0) & (d1 >= 0) & (d3 <= 0)
        rx = jnp.where(m_ab, ax + v_ab * abx, rx)
        ry = jnp.where(m_ab, ay + v_ab * aby, ry)
        rz = jnp.where(m_ab, az + v_ab * abz, rz)

        m_c = (d6 >= 0) & (d5 <= d6)
        rx = jnp.where(m_c, cx, rx)
        ry = jnp.where(m_c, cy, ry)
        rz = jnp.where(m_c, cz, rz)
        m_b = (d3 >= 0) & (d4 <= d3)
        rx = jnp.where(m_b, bx, rx)
        ry = jnp.where(m_b, by, ry)
        rz = jnp.where(m_b, bz, rz)
        m_a = (d1 <= 0) & (d2 <= 0)
        rx = jnp.where(m_a, ax, rx)
        ry = jnp.where(m_a, ay, ry)
        rz = jnp.where(m_a, az, rz)

        ex = px - rx
        ey = py - ry
        ez = pz - rz
        dist = ex * ex + ey * ey + ez * ez

        m = jnp.min(dist, axis=1, keepdims=True)
        iota = jax.lax.broadcasted_iota(jnp.int32, (NT, FC), 1)
        idx = jnp.min(jnp.where(dist == m, iota, jnp.int32(2**30)),
                      axis=1, keepdims=True) + jnp.int32(j * FC)
        take = m < best_d
        best_i = jnp.where(take, idx, best_i)
        best_d = jnp.where(take, m, best_d)

    dist_ref[:, :] = best_d
    assoc_ref[:, :] = best_i

    n = pl.program_id(0)

    @pl.when(n == 0)
    def _():
        loss_ref[:, :] = jnp.zeros((1, 1), jnp.float32)

    loss_ref[:, :] += jnp.sum(best_d, axis=0, keepdims=True)


def kernel(verts, faces, points):
    P = points.shape[0]
    F = faces.shape[0]
    tri = jnp.take(verts, faces.astype(jnp.int32), axis=0)
    tri_t = tri.reshape(F, 9).T

    grid = (P // NT,)
    dist2d, assoc2d, loss2d = pl.pallas_call(
        _mesh_sdf_kernel,
        grid=grid,
        in_specs=[
            pl.BlockSpec((9, F), lambda n: (0, 0)),
            pl.BlockSpec((NT, 3), lambda n: (n, 0)),
        ],
        out_specs=[
            pl.BlockSpec((NT, 1), lambda n: (n, 0)),
            pl.BlockSpec((NT, 1), lambda n: (n, 0)),
            pl.BlockSpec((1, 1), lambda n: (0, 0)),
        ],
        out_shape=[
            jax.ShapeDtypeStruct((P, 1), jnp.float32),
            jax.ShapeDtypeStruct((P, 1), jnp.int32),
            jax.ShapeDtypeStruct((1, 1), jnp.float32),
        ],
    )(tri_t, points)

    dist = dist2d.reshape(P)
    assoc = assoc2d.reshape(P).astype(jnp.int64)
    loss = loss2d[0, 0] * (1000.0 / P)
    return loss, dist, assoc


# speedup vs baseline: 3.0045x; 1.0976x over previous
import jax
import jax.numpy as jnp
from jax.experimental import pallas as pl
from jax.experimental.pallas import tpu as pltpu

EPS_ = 1e-12
NT = 256
FC = 256


def _mesh_sdf_kernel(tri_ref, pts_ref, dist_ref, assoc_ref, loss_ref):
    F = tri_ref.shape[1]
    n_chunks = F // FC

    px = pts_ref[:, 0:1]
    py = pts_ref[:, 1:2]
    pz = pts_ref[:, 2:3]

    best_d = jnp.full((NT, 1), jnp.inf, dtype=jnp.float32)
    best_i = jnp.zeros((NT, 1), dtype=jnp.int32)

    for j in range(n_chunks):
        sl = slice(j * FC, (j + 1) * FC)
        ax = tri_ref[0:1, sl]
        ay = tri_ref[1:2, sl]
        az = tri_ref[2:3, sl]
        bx = tri_ref[3:4, sl]
        by = tri_ref[4:5, sl]
        bz = tri_ref[5:6, sl]
        cx = tri_ref[6:7, sl]
        cy = tri_ref[7:8, sl]
        cz = tri_ref[8:9, sl]

        abx = bx - ax
        aby = by - ay
        abz = bz - az
        acx = cx - ax
        acy = cy - ay
        acz = cz - az
        bcx = cx - bx
        bcy = cy - by
        bcz = cz - bz

        apx = px - ax
        apy = py - ay
        apz = pz - az
        d1 = abx * apx + aby * apy + abz * apz
        d2 = acx * apx + acy * apy + acz * apz
        bpx = px - bx
        bpy = py - by
        bpz = pz - bz
        d3 = abx * bpx + aby * bpy + abz * bpz
        d4 = acx * bpx + acy * bpy + acz * bpz
        cpx = px - cx
        cpy = py - cy
        cpz = pz - cz
        d5 = abx * cpx + aby * cpy + abz * cpz
        d6 = acx * cpx + acy * cpy + acz * cpz

        vc = d1 * d4 - d3 * d2
        vb = d5 * d2 - d1 * d6
        va = d3 * d6 - d5 * d4

        denom = (va + vb) + vc
        denom = jnp.where(jnp.abs(denom) < EPS_, EPS_, denom)
        v = vb / denom
        w = vc / denom
        rx = ax + v * abx + w * acx
        ry = ay + v * aby + w * acy
        rz = az + v * abz + w * acz

        e45 = d4 - d3
        e56 = d5 - d6
        w_bc = e45 / jnp.maximum(e45 + e56, EPS_)
        m_bc = (va <= 0) & (e45 >= 0) & (e56 >= 0)
        rx = jnp.where(m_bc, bx + w_bc * bcx, rx)
        ry = jnp.where(m_bc, by + w_bc * bcy, ry)
        rz = jnp.where(m_bc, bz + w_bc * bcz, rz)

        v_ac = d2 / jnp.maximum(d2 - d6, EPS_)
        m_ac = (vb <= 0) & (d2 >= 0) & (d6 <= 0)
        rx = jnp.where(m_ac, ax + v_ac * acx, rx)
        ry = jnp.where(m_ac, ay + v_ac * acy, ry)
        rz = jnp.where(m_ac, az + v_ac * acz, rz)

        v_ab = d1 / jnp.maximum(d1 - d3, EPS_)
        m_ab = (vc <= 0) & (d1 >= 0) & (d3 <= 0)
        rx = jnp.where(m_ab, ax + v_ab * abx, rx)
        ry = jnp.where(m_ab, ay + v_ab * aby, ry)
        rz = jnp.where(m_ab, az + v_ab * abz, rz)

        m_c = (d6 >= 0) & (d5 <= d6)
        rx = jnp.where(m_c, cx, rx)
        ry = jnp.where(m_c, cy, ry)
        rz = jnp.where(m_c, cz, rz)
        m_b = (d3 >= 0) & (d4 <= d3)
        rx = jnp.where(m_b, bx, rx)
        ry = jnp.where(m_b, by, ry)
        rz = jnp.where(m_b, bz, rz)
        m_a = (d1 <= 0) & (d2 <= 0)
        rx = jnp.where(m_a, ax, rx)
        ry = jnp.where(m_a, ay, ry)
        rz = jnp.where(m_a, az, rz)

        ex = px - rx
        ey = py - ry
        ez = pz - rz
        dist = ex * ex + ey * ey + ez * ez

        m = jnp.min(dist, axis=1, keepdims=True)
        iota = jax.lax.broadcasted_iota(jnp.int32, (NT, FC), 1)
        idx = jnp.min(jnp.where(dist == m, iota, jnp.int32(2**30)),
                      axis=1, keepdims=True) + jnp.int32(j * FC)
        take = m < best_d
        best_i = jnp.where(take, idx, best_i)
        best_d = jnp.where(take, m, best_d)

    dist_ref[:, :] = best_d
    assoc_ref[:, :] = best_i

    n = pl.program_id(0)

    @pl.when(n == 0)
    def _():
        loss_ref[:, :] = jnp.zeros((1, 1), jnp.float32)

    loss_ref[:, :] += jnp.sum(best_d, axis=0, keepdims=True)


def kernel(verts, faces, points):
    P = points.shape[0]
    F = faces.shape[0]
    tri = jnp.take(verts, faces.astype(jnp.int32), axis=0)
    tri_t = tri.reshape(F, 9).T

    grid = (P // NT,)
    dist2d, assoc2d, loss2d = pl.pallas_call(
        _mesh_sdf_kernel,
        grid=grid,
        in_specs=[
            pl.BlockSpec((9, F), lambda n: (0, 0)),
            pl.BlockSpec((NT, 3), lambda n: (n, 0)),
        ],
        out_specs=[
            pl.BlockSpec((NT, 1), lambda n: (n, 0)),
            pl.BlockSpec((NT, 1), lambda n: (n, 0)),
            pl.BlockSpec((1, 1), lambda n: (0, 0)),
        ],
        out_shape=[
            jax.ShapeDtypeStruct((P, 1), jnp.float32),
            jax.ShapeDtypeStruct((P, 1), jnp.int32),
            jax.ShapeDtypeStruct((1, 1), jnp.float32),
        ],
    )(tri_t, points)

    dist = dist2d.reshape(P)
    assoc = assoc2d.reshape(P).astype(jnp.int64)
    loss = loss2d[0, 0] * (1000.0 / P)
    return loss, dist, assoc


# speedup vs baseline: 5.1213x; 1.7045x over previous
import jax
import jax.numpy as jnp
from jax.experimental import pallas as pl
from jax.experimental.pallas import tpu as pltpu

EPS_ = 1e-12
NT = 256
FC = 256


def _mesh_sdf_kernel(tri_ref, pts_ref, dist_ref, assoc_ref, loss_ref):
    F = tri_ref.shape[1]
    n_chunks = F // FC

    px = pts_ref[:, 0:1]
    py = pts_ref[:, 1:2]
    pz = pts_ref[:, 2:3]

    best_d = jnp.full((NT, 1), jnp.inf, dtype=jnp.float32)
    best_i = jnp.zeros((NT, 1), dtype=jnp.int32)

    for j in range(n_chunks):
        sl = slice(j * FC, (j + 1) * FC)
        ax = tri_ref[0:1, sl]
        ay = tri_ref[1:2, sl]
        az = tri_ref[2:3, sl]
        bx = tri_ref[3:4, sl]
        by = tri_ref[4:5, sl]
        bz = tri_ref[5:6, sl]
        cx = tri_ref[6:7, sl]
        cy = tri_ref[7:8, sl]
        cz = tri_ref[8:9, sl]

        abx = bx - ax
        aby = by - ay
        abz = bz - az
        acx = cx - ax
        acy = cy - ay
        acz = cz - az
        bcx = cx - bx
        bcy = cy - by
        bcz = cz - bz

        apx = px - ax
        apy = py - ay
        apz = pz - az
        d1 = abx * apx + aby * apy + abz * apz
        d2 = acx * apx + acy * apy + acz * apz
        bpx = px - bx
        bpy = py - by
        bpz = pz - bz
        d3 = abx * bpx + aby * bpy + abz * bpz
        d4 = acx * bpx + acy * bpy + acz * bpz
        cpx = px - cx
        cpy = py - cy
        cpz = pz - cz
        d5 = abx * cpx + aby * cpy + abz * cpz
        d6 = acx * cpx + acy * cpy + acz * cpz

        vc = d1 * d4 - d3 * d2
        vb = d5 * d2 - d1 * d6
        va = d3 * d6 - d5 * d4

        denom = (va + vb) + vc
        denom = jnp.where(jnp.abs(denom) < EPS_, EPS_, denom)
        v = vb / denom
        w = vc / denom
        rx = ax + v * abx + w * acx
        ry = ay + v * aby + w * acy
        rz = az + v * abz + w * acz

        e45 = d4 - d3
        e56 = d5 - d6
        w_bc = e45 / jnp.maximum(e45 + e56, EPS_)
        m_bc = (va <= 0) & (e45 >= 0) & (e56 >= 0)
        rx = jnp.where(m_bc, bx + w_bc * bcx, rx)
        ry = jnp.where(m_bc, by + w_bc * bcy, ry)
        rz = jnp.where(m_bc, bz + w_bc * bcz, rz)

        v_ac = d2 / jnp.maximum(d2 - d6, EPS_)
        m_ac = (vb <= 0) & (d2 >= 0) & (d6 <= 0)
        rx = jnp.where(m_ac, ax + v_ac * acx, rx)
        ry = jnp.where(m_ac, ay + v_ac * acy, ry)
        rz = jnp.where(m_ac, az + v_ac * acz, rz)

        v_ab = d1 / jnp.maximum(d1 - d3, EPS_)
        m_ab = (vc <= 0) & (d1 >= 0) & (d3 <= 0)
        rx = jnp.where(m_ab, ax + v_ab * abx, rx)
        ry = jnp.where(m_ab, ay + v_ab * aby, ry)
        rz = jnp.where(m_ab, az + v_ab * abz, rz)

        m_c = (d6 >= 0) & (d5 <= d6)
        rx = jnp.where(m_c, cx, rx)
        ry = jnp.where(m_c, cy, ry)
        rz = jnp.where(m_c, cz, rz)
        m_b = (d3 >= 0) & (d4 <= d3)
        rx = jnp.where(m_b, bx, rx)
        ry = jnp.where(m_b, by, ry)
        rz = jnp.where(m_b, bz, rz)
        m_a = (d1 <= 0) & (d2 <= 0)
        rx = jnp.where(m_a, ax, rx)
        ry = jnp.where(m_a, ay, ry)
        rz = jnp.where(m_a, az, rz)

        ex = px - rx
        ey = py - ry
        ez = pz - rz
        dist = ex * ex + ey * ey + ez * ez

        m = jnp.min(dist, axis=1, keepdims=True)
        iota = jax.lax.broadcasted_iota(jnp.int32, (NT, FC), 1)
        idx = jnp.min(jnp.where(dist == m, iota, jnp.int32(2**30)),
                      axis=1, keepdims=True) + jnp.int32(j * FC)
        take = m < best_d
        best_i = jnp.where(take, idx, best_i)
        best_d = jnp.where(take, m, best_d)

    dist_ref[:, :] = best_d
    assoc_ref[:, :] = best_i

    n = pl.program_id(0)

    @pl.when(n == 0)
    def _():
        loss_ref[:, :] = jnp.zeros((1, 1), jnp.float32)

    loss_ref[:, :] += jnp.sum(best_d, axis=0, keepdims=True)


def _pairwise_call(tri_t, points):
    P = points.shape[0]
    F = tri_t.shape[1]
    grid = (P // NT,)
    return pl.pallas_call(
        _mesh_sdf_kernel,
        grid=grid,
        in_specs=[
            pl.BlockSpec((9, F), lambda n: (0, 0)),
            pl.BlockSpec((NT, 3), lambda n: (n, 0)),
        ],
        out_specs=[
            pl.BlockSpec((NT, 1), lambda n: (n, 0)),
            pl.BlockSpec((NT, 1), lambda n: (n, 0)),
            pl.BlockSpec((1, 1), lambda n: (0, 0)),
        ],
        out_shape=[
            jax.ShapeDtypeStruct((P, 1), jnp.float32),
            jax.ShapeDtypeStruct((P, 1), jnp.int32),
            jax.ShapeDtypeStruct((1, 1), jnp.float32),
        ],
    )(tri_t, points)


def kernel(verts, faces, points):
    P = points.shape[0]
    F = faces.shape[0]
    tri = jnp.take(verts, faces.astype(jnp.int32), axis=0)
    tri_t = tri.reshape(F, 9).T

    devs = jax.devices()
    n_dev = max(n for n in range(1, len(devs) + 1) if (P // NT) % n == 0)

    if n_dev > 1:
        from jax.experimental.shard_map import shard_map
        from jax.sharding import Mesh, PartitionSpec as Ps
        import numpy as np

        mesh = Mesh(np.array(devs[:n_dev]), ("d",))

        def _shard_fn(tri_rep, pts_shard):
            d2, a2, l2 = _pairwise_call(tri_rep, pts_shard)
            return d2, a2, jax.lax.psum(l2, "d")

        dist2d, assoc2d, loss2d = shard_map(
            _shard_fn, mesh=mesh,
            in_specs=(Ps(None, None), Ps("d", None)),
            out_specs=(Ps("d", None), Ps("d", None), Ps(None, None)),
            check_rep=False,
        )(tri_t, points)
    else:
        dist2d, assoc2d, loss2d = _pairwise_call(tri_t, points)

    dist = dist2d.reshape(P)
    assoc = assoc2d.reshape(P).astype(jnp.int64)
    loss = loss2d[0, 0] * (1000.0 / P)
    return loss, dist, assoc


# speedup vs baseline: 5.3893x; 1.0523x over previous
import functools

import jax
import jax.numpy as jnp
from jax import lax
from jax.experimental import pallas as pl
from jax.experimental.pallas import tpu as pltpu
from jax.experimental.pallas import tpu_sc as plsc

EPS_ = 1e-12
NT = 256
FC = 256


def _mesh_sdf_kernel(tri_ref, pts_ref, dist_ref, assoc_ref, loss_ref):
    F = tri_ref.shape[1]
    n_chunks = F // FC

    px = pts_ref[:, 0:1]
    py = pts_ref[:, 1:2]
    pz = pts_ref[:, 2:3]

    best_d = jnp.full((NT, 1), jnp.inf, dtype=jnp.float32)
    best_i = jnp.zeros((NT, 1), dtype=jnp.int32)

    for j in range(n_chunks):
        sl = slice(j * FC, (j + 1) * FC)
        ax = tri_ref[0:1, sl]
        ay = tri_ref[1:2, sl]
        az = tri_ref[2:3, sl]
        bx = tri_ref[3:4, sl]
        by = tri_ref[4:5, sl]
        bz = tri_ref[5:6, sl]
        cx = tri_ref[6:7, sl]
        cy = tri_ref[7:8, sl]
        cz = tri_ref[8:9, sl]

        abx = bx - ax
        aby = by - ay
        abz = bz - az
        acx = cx - ax
        acy = cy - ay
        acz = cz - az
        bcx = cx - bx
        bcy = cy - by
        bcz = cz - bz

        apx = px - ax
        apy = py - ay
        apz = pz - az
        d1 = abx * apx + aby * apy + abz * apz
        d2 = acx * apx + acy * apy + acz * apz
        bpx = px - bx
        bpy = py - by
        bpz = pz - bz
        d3 = abx * bpx + aby * bpy + abz * bpz
        d4 = acx * bpx + acy * bpy + acz * bpz
        cpx = px - cx
        cpy = py - cy
        cpz = pz - cz
        d5 = abx * cpx + aby * cpy + abz * cpz
        d6 = acx * cpx + acy * cpy + acz * cpz

        vc = d1 * d4 - d3 * d2
        vb = d5 * d2 - d1 * d6
        va = d3 * d6 - d5 * d4

        denom = (va + vb) + vc
        denom = jnp.where(jnp.abs(denom) < EPS_, EPS_, denom)
        v = vb / denom
        w = vc / denom
        rx = ax + v * abx + w * acx
        ry = ay + v * aby + w * acy
        rz = az + v * abz + w * acz

        e45 = d4 - d3
        e56 = d5 - d6
        w_bc = e45 / jnp.maximum(e45 + e56, EPS_)
        m_bc = (va <= 0) & (e45 >= 0) & (e56 >= 0)
        rx = jnp.where(m_bc, bx + w_bc * bcx, rx)
        ry = jnp.where(m_bc, by + w_bc * bcy, ry)
        rz = jnp.where(m_bc, bz + w_bc * bcz, rz)

        v_ac = d2 / jnp.maximum(d2 - d6, EPS_)
        m_ac = (vb <= 0) & (d2 >= 0) & (d6 <= 0)
        rx = jnp.where(m_ac, ax + v_ac * acx, rx)
        ry = jnp.where(m_ac, ay + v_ac * acy, ry)
        rz = jnp.where(m_ac, az + v_ac * acz, rz)

        v_ab = d1 / jnp.maximum(d1 - d3, EPS_)
        m_ab = (vc <= 0) & (d1 >= 0) & (d3 <= 0)
        rx = jnp.where(m_ab, ax + v_ab * abx, rx)
        ry = jnp.where(m_ab, ay + v_ab * aby, ry)
        rz = jnp.where(m_ab, az + v_ab * abz, rz)

        m_c = (d6 >= 0) & (d5 <= d6)
        rx = jnp.where(m_c, cx, rx)
        ry = jnp.where(m_c, cy, ry)
        rz = jnp.where(m_c, cz, rz)
        m_b = (d3 >= 0) & (d4 <= d3)
        rx = jnp.where(m_b, bx, rx)
        ry = jnp.where(m_b, by, ry)
        rz = jnp.where(m_b, bz, rz)
        m_a = (d1 <= 0) & (d2 <= 0)
        rx = jnp.where(m_a, ax, rx)
        ry = jnp.where(m_a, ay, ry)
        rz = jnp.where(m_a, az, rz)

        ex = px - rx
        ey = py - ry
        ez = pz - rz
        dist = ex * ex + ey * ey + ez * ez

        m = jnp.min(dist, axis=1, keepdims=True)
        iota = jax.lax.broadcasted_iota(jnp.int32, (NT, FC), 1)
        idx = jnp.min(jnp.where(dist == m, iota, jnp.int32(2**30)),
                      axis=1, keepdims=True) + jnp.int32(j * FC)
        take = m < best_d
        best_i = jnp.where(take, idx, best_i)
        best_d = jnp.where(take, m, best_d)

    dist_ref[:, :] = best_d
    assoc_ref[:, :] = best_i

    n = pl.program_id(0)

    @pl.when(n == 0)
    def _():
        loss_ref[:, :] = jnp.zeros((1, 1), jnp.float32)

    loss_ref[:, :] += jnp.sum(best_d, axis=0, keepdims=True)


def _sc_gather_rows(verts16, idx):
    B = idx.shape[0]
    mesh = plsc.VectorSubcoreMesh(core_axis_name="c", subcore_axis_name="s")
    nw = mesh.num_cores * mesh.num_subcores
    b_per_w = B // nw

    @functools.partial(
        pl.kernel,
        mesh=mesh,
        out_type=jax.ShapeDtypeStruct((B, 16), jnp.float32),
        scratch_types=[
            pltpu.VMEM((b_per_w,), jnp.int32),
            pltpu.VMEM((b_per_w, 16), jnp.float32),
            pltpu.SemaphoreType.DMA,
        ],
        compiler_params=pltpu.CompilerParams(use_tc_tiling_on_sc=False),
    )
    def k(table_hbm, idx_hbm, out_hbm, idx_v, rows_v, sem):
        wid = lax.axis_index("s") * mesh.num_cores + lax.axis_index("c")
        base = wid * b_per_w
        pltpu.sync_copy(idx_hbm.at[pl.ds(base, b_per_w)], idx_v)
        pltpu.async_copy(table_hbm.at[idx_v], rows_v, sem).wait()
        pltpu.sync_copy(rows_v, out_hbm.at[pl.ds(base, b_per_w)])

    return k(verts16, idx)


def _pairwise_call(tri_t, points):
    P = points.shape[0]
    F = tri_t.shape[1]
    grid = (P // NT,)
    return pl.pallas_call(
        _mesh_sdf_kernel,
        grid=grid,
        in_specs=[
            pl.BlockSpec((9, F), lambda n: (0, 0)),
            pl.BlockSpec((NT, 3), lambda n: (n, 0)),
        ],
        out_specs=[
            pl.BlockSpec((NT, 1), lambda n: (n, 0)),
            pl.BlockSpec((NT, 1), lambda n: (n, 0)),
            pl.BlockSpec((1, 1), lambda n: (0, 0)),
        ],
        out_shape=[
            jax.ShapeDtypeStruct((P, 1), jnp.float32),
            jax.ShapeDtypeStruct((P, 1), jnp.int32),
            jax.ShapeDtypeStruct((1, 1), jnp.float32),
        ],
    )(tri_t, points)


def kernel(verts, faces, points):
    P = points.shape[0]
    F = faces.shape[0]
    verts16 = jnp.pad(verts, ((0, 0), (0, 13)))
    idx_flat = faces.astype(jnp.int32).reshape(3 * F)

    def _build_tri_t(v16, idx):
        tri16 = _sc_gather_rows(v16, idx)
        return tri16[:, :3].reshape(F, 9).T

    devs = jax.devices()
    n_dev = max(n for n in range(1, len(devs) + 1) if (P // NT) % n == 0)

    if n_dev > 1:
        from jax.experimental.shard_map import shard_map
        from jax.sharding import Mesh, PartitionSpec as Ps
        import numpy as np

        mesh = Mesh(np.array(devs[:n_dev]), ("d",))

        def _shard_fn(v16, idx, pts_shard):
            d2, a2, l2 = _pairwise_call(_build_tri_t(v16, idx), pts_shard)
            return d2, a2, jax.lax.psum(l2, "d")

        dist2d, assoc2d, loss2d = shard_map(
            _shard_fn, mesh=mesh,
            in_specs=(Ps(None, None), Ps(None), Ps("d", None)),
            out_specs=(Ps("d", None), Ps("d", None), Ps(None, None)),
            check_rep=False,
        )(verts16, idx_flat, points)
    else:
        dist2d, assoc2d, loss2d = _pairwise_call(
            _build_tri_t(verts16, idx_flat), points)

    dist = dist2d.reshape(P)
    assoc = assoc2d.reshape(P).astype(jnp.int64)
    loss = loss2d[0, 0] * (1000.0 / P)
    return loss, dist, assoc
